# baseline (speedup 1.0000x reference)
import functools

import jax
import jax.numpy as jnp
from jax import lax
from jax.experimental import pallas as pl
from jax.experimental.pallas import tpu as pltpu
from jax.experimental.pallas import tpu_sc as plsc

NC = 2
NS = 16
LANES = 16
DEPTH = 4


def _emb_body(rows_per_w, n_rows, S, D,
              ids_hbm, tids_hbm, tok_hbm, typ_hbm, out_hbm,
              typ_v,
              ix0, ix1, ix2, ix3, tx0, tx1, tx2, tx3,
              rows0, rows1, rows2, rows3,
              g0, g1, g2, g3, i0, i1, i2, i3, s0, s1, s2, s3):
    ixb = [ix0, ix1, ix2, ix3]
    txb = [tx0, tx1, tx2, tx3]
    rows = [rows0, rows1, rows2, rows3]
    gsem = [g0, g1, g2, g3]
    isem = [i0, i1, i2, i3]
    ssem = [s0, s1, s2, s3]
    wid = lax.axis_index("s") * NC + lax.axis_index("c")
    w_lo = wid * rows_per_w
    nd = D // LANES
    n_full = (S // LANES) * LANES

    pltpu.sync_copy(typ_hbm, typ_v)
    row0 = [typ_v[pl.ds(d * LANES, LANES)] for d in range(nd)]
    diff = [typ_v[pl.ds(D + d * LANES, LANES)] - row0[d] for d in range(nd)]

    def idx_copy(r, b):
        pltpu.async_copy(ids_hbm.at[w_lo + r], ixb[b], isem[b])
        pltpu.async_copy(tids_hbm.at[w_lo + r], txb[b], isem[b])

    def idx_wait(r, b):
        pltpu.make_async_copy(ids_hbm.at[w_lo + r], ixb[b], isem[b]).wait()
        pltpu.make_async_copy(tids_hbm.at[w_lo + r], txb[b], isem[b]).wait()

    def gather_parts(b):
        h = (S // 2 + 7) & ~7
        yield (tok_hbm.at[ixb[b].at[pl.ds(0, h)]],
               rows[b].at[pl.ds(0, h)], gsem[b])
        yield (tok_hbm.at[ixb[b].at[pl.ds(h, S - h)]],
               rows[b].at[pl.ds(h, S - h)], gsem[b])

    def gather(b):
        for src, dst, sem in gather_parts(b):
            pltpu.async_copy(src, dst, sem)

    def gather_wait(b):
        for src, dst, sem in gather_parts(b):
            pltpu.make_async_copy(src, dst, sem).wait()

    def out_dst(r):
        return out_hbm.at[w_lo + r].at[:, pl.ds(0, D)]

    def add_block(b, t0, tid16, lanes):
        for l in lanes:
            tidf = tid16[l]
            for d in range(nd):
                sl = pl.ds(d * LANES, LANES)
                rows[b][t0 + l, sl] = rows[b][t0 + l, sl] + (
                    row0[d] + tidf * diff[d])

    pltpu.sync_copy(ids_hbm.at[w_lo], ixb[0])
    pltpu.sync_copy(tids_hbm.at[w_lo], txb[0])
    pltpu.sync_copy(ids_hbm.at[w_lo + 1], ixb[1])
    pltpu.sync_copy(tids_hbm.at[w_lo + 1], txb[1])
    idx_copy(2, 2)
    gather(0)
    gather(1)

    def super_body(si, carry):
        for p in range(DEPTH):
            r = si * DEPTH + p
            b = p
            b2 = (p + 2) % DEPTH
            b3 = (p + 3) % DEPTH

            @pl.when(r + 3 < n_rows)
            def _():
                idx_copy(r + 3, b3)

            @pl.when(r >= 2)
            def _():
                pltpu.make_async_copy(rows[b2], out_dst(0), ssem[b2]).wait()

            @pl.when(r + 2 < n_rows)
            def _():
                idx_wait(r + 2, b2)
                gather(b2)

            gather_wait(b)

            @plsc.parallel_loop(0, n_full, step=LANES, unroll=2)
            def add_body(t0):
                tid16 = txb[b][pl.ds(t0, LANES)].astype(jnp.float32)
                add_block(b, t0, tid16, range(LANES))

            if n_full < S:
                t0 = S - LANES
                tid16 = txb[b][pl.ds(t0, LANES)].astype(jnp.float32)
                add_block(b, t0, tid16, range(n_full - t0, LANES))

            pltpu.async_copy(rows[b], out_dst(r), ssem[b])
        return carry

    lax.fori_loop(0, n_rows // DEPTH, super_body, 0)

    for b in ((n_rows - 2) % DEPTH, (n_rows - 1) % DEPTH):
        pltpu.make_async_copy(rows[b], out_dst(0), ssem[b]).wait()


def kernel(input_ids, type_ids, token_table, type_table):
    B, S = input_ids.shape
    V, D = token_table.shape
    NW = NC * NS
    rows_per_w = B // NW
    n_rows = rows_per_w
    assert rows_per_w * NW == B and n_rows % DEPTH == 0

    ids = input_ids.astype(jnp.int32)
    tids = type_ids.astype(jnp.int32)
    tok = jax.lax.optimization_barrier(token_table.reshape(V * D))
    tok = tok.reshape(V, D)
    typ = type_table.reshape(2 * D)

    mesh = plsc.VectorSubcoreMesh(
        core_axis_name="c", subcore_axis_name="s",
        num_cores=NC, num_subcores=NS)

    emb = functools.partial(
        pl.kernel,
        out_type=jax.ShapeDtypeStruct((B, S, 2 * D), jnp.float32),
        mesh=mesh,
        scratch_types=[
            pltpu.VMEM((2 * D,), jnp.float32),
        ] + [pltpu.VMEM((S,), jnp.int32)] * (2 * DEPTH)
          + [pltpu.VMEM((S, D), jnp.float32)] * DEPTH
          + [pltpu.SemaphoreType.DMA] * (3 * DEPTH),
        compiler_params=pltpu.CompilerParams(use_tc_tiling_on_sc=False),
    )(functools.partial(_emb_body, rows_per_w, n_rows, S, D))

    wide = emb(ids, tids, tok, typ)
    return wide[:, :, :D]
